# Initial kernel scaffold; baseline (speedup 1.0000x reference)
import functools

import jax
import jax.numpy as jnp
from jax import lax
from jax.experimental import pallas as pl
from jax.experimental.pallas import tpu as pltpu
from jax.experimental.pallas import tpu_sc as plsc

B = 4096
VOCAB = 100000
WIDE_DIM = 8
DEEP_DIM = 26
N_WIDE = 2
WD = 16
N_DEEP = 6
DD = 32
N_TAB = N_WIDE + N_DEEP
DEEP_RAW = DEEP_DIM - N_DEEP
WIDE_RAW = WIDE_DIM - N_WIDE
DEEP_IN = N_DEEP * DD + DEEP_RAW
WIDE_OUT = N_WIDE * WD + WIDE_RAW
H0, H1 = 256, 128
ACTION_DIM = 64

NC, NS = 2, 16
NW = NC * NS
BPW = B // NW


def _build_gather():
    mesh = plsc.VectorSubcoreMesh(
        core_axis_name="c", subcore_axis_name="s", num_cores=NC, num_subcores=NS
    )
    out_type = (
        [jax.ShapeDtypeStruct((B, WD), jnp.float32)] * N_WIDE
        + [jax.ShapeDtypeStruct((B, DD), jnp.float32)] * N_DEEP
    )
    scratch = [
        pltpu.VMEM((BPW,), jnp.int32),
        pltpu.VMEM((BPW, WD), jnp.float32),
        pltpu.VMEM((BPW, DD), jnp.float32),
        pltpu.SemaphoreType.DMA,
    ]

    @functools.partial(pl.kernel, mesh=mesh, out_type=out_type, scratch_types=scratch)
    def gather_k(idx_hbm, ew0, ew1, ed0, ed1, ed2, ed3, ed4, ed5,
                 ow0, ow1, od0, od1, od2, od3, od4, od5,
                 idx_v, rows_w, rows_d, sem):
        wid = lax.axis_index("s") * NC + lax.axis_index("c")
        base = wid * BPW
        tables = [ew0, ew1, ed0, ed1, ed2, ed3, ed4, ed5]
        outs = [ow0, ow1, od0, od1, od2, od3, od4, od5]
        for t in range(N_TAB):
            rows = rows_w if t < N_WIDE else rows_d
            pltpu.sync_copy(idx_hbm.at[pl.ds(t * B + base, BPW)], idx_v)
            pltpu.async_copy(tables[t].at[idx_v], rows, sem).wait()
            pltpu.sync_copy(rows, outs[t].at[pl.ds(base, BPW)])

    return gather_k


_gather = _build_gather()

BLK = 512
GRID = B // BLK


def _mlp_body(xr, w0e, w1e, d0, d1, d2, d3, d4, d5,
              w0t, b0, w1t, b1, wltw, wlth, bl, out):
    deep_cat = jnp.concatenate(
        [d0[...], d1[...], d2[...], d3[...], d4[...], d5[...],
         xr[:, WIDE_DIM + N_DEEP:]], axis=1)
    h = jnp.dot(deep_cat, w0t[...], preferred_element_type=jnp.float32) + b0[...]
    h = jnp.maximum(h, 0.0)
    h = jnp.dot(h, w1t[...], preferred_element_type=jnp.float32) + b1[...]
    h = jnp.maximum(h, 0.0)
    wide_cat = jnp.concatenate(
        [w0e[...], w1e[...], xr[:, N_WIDE:WIDE_DIM]], axis=1)
    out[...] = (
        jnp.dot(wide_cat, wltw[...], preferred_element_type=jnp.float32)
        + jnp.dot(h, wlth[...], preferred_element_type=jnp.float32)
        + bl[...]
    )


def _row_spec(d):
    return pl.BlockSpec((BLK, d), lambda i: (i, 0))


def _full_spec(shape):
    return pl.BlockSpec(shape, lambda i: (0,) * len(shape))


def _mlp(xr, w0e, w1e, ds, w0t, b0, w1t, b1, wltw, wlth, bl, interpret=False):
    in_specs = (
        [_row_spec(WIDE_DIM + DEEP_DIM), _row_spec(WD), _row_spec(WD)]
        + [_row_spec(DD)] * N_DEEP
        + [_full_spec(w0t.shape), _full_spec(b0.shape), _full_spec(w1t.shape),
           _full_spec(b1.shape), _full_spec(wltw.shape), _full_spec(wlth.shape),
           _full_spec(bl.shape)]
    )
    return pl.pallas_call(
        _mlp_body,
        grid=(GRID,),
        in_specs=in_specs,
        out_specs=_row_spec(ACTION_DIM),
        out_shape=jax.ShapeDtypeStruct((B, ACTION_DIM), jnp.float32),
        interpret=interpret,
    )(xr, w0e, w1e, *ds, w0t, b0, w1t, b1, wltw, wlth, bl)


def kernel(x, Ew0, Ew1, Ed0, Ed1, Ed2, Ed3, Ed4, Ed5, W0, b0, W1, b1, Wl, bl):
    xi = x.astype(jnp.int32)
    cols = [0, 1] + [WIDE_DIM + i for i in range(N_DEEP)]
    idx_flat = jnp.concatenate([xi[:, c] for c in cols], axis=0)
    gathered = _gather(idx_flat, Ew0, Ew1, Ed0, Ed1, Ed2, Ed3, Ed4, Ed5)
    w0e, w1e = gathered[0], gathered[1]
    ds = gathered[2:]
    wlt = Wl.T
    return _mlp(
        x, w0e, w1e, ds,
        W0.T, b0[None, :], W1.T, b1[None, :],
        wlt[:WIDE_OUT], wlt[WIDE_OUT:], bl[None, :],
    )


# speedup vs baseline: 1.0430x; 1.0430x over previous
import functools

import jax
import jax.numpy as jnp
from jax import lax
from jax.experimental import pallas as pl
from jax.experimental.pallas import tpu as pltpu
from jax.experimental.pallas import tpu_sc as plsc

B = 4096
VOCAB = 100000
WIDE_DIM = 8
DEEP_DIM = 26
N_WIDE = 2
WD = 16
N_DEEP = 6
DD = 32
N_TAB = N_WIDE + N_DEEP
DEEP_RAW = DEEP_DIM - N_DEEP
WIDE_RAW = WIDE_DIM - N_WIDE
DEEP_IN = N_DEEP * DD + DEEP_RAW
WIDE_OUT = N_WIDE * WD + WIDE_RAW
H0, H1 = 256, 128
ACTION_DIM = 64

NC, NS = 2, 16
NW = NC * NS
BPW = B // NW


@functools.cache
def _build_gather():
    mesh = plsc.VectorSubcoreMesh(
        core_axis_name="c", subcore_axis_name="s", num_cores=NC, num_subcores=NS
    )
    out_type = (
        [jax.ShapeDtypeStruct((B, WD), jnp.float32)] * N_WIDE
        + [jax.ShapeDtypeStruct((B, DD), jnp.float32)] * N_DEEP
    )
    scratch = [
        pltpu.VMEM((BPW,), jnp.int32),
        pltpu.VMEM((BPW, WD), jnp.float32),
        pltpu.VMEM((BPW, DD), jnp.float32),
        pltpu.SemaphoreType.DMA,
    ]

    @functools.partial(
        pl.kernel, mesh=mesh, out_type=out_type, scratch_types=scratch,
        compiler_params=pltpu.CompilerParams(use_tc_tiling_on_sc=False))
    def gather_k(idx_hbm, ew0, ew1, ed0, ed1, ed2, ed3, ed4, ed5,
                 ow0, ow1, od0, od1, od2, od3, od4, od5,
                 idx_v, rows_w, rows_d, sem):
        wid = lax.axis_index("s") * NC + lax.axis_index("c")
        base = wid * BPW
        tables = [ew0, ew1, ed0, ed1, ed2, ed3, ed4, ed5]
        outs = [ow0, ow1, od0, od1, od2, od3, od4, od5]
        for t in range(N_TAB):
            rows = rows_w if t < N_WIDE else rows_d
            pltpu.sync_copy(idx_hbm.at[pl.ds(t * B + base, BPW)], idx_v)
            pltpu.async_copy(tables[t].at[idx_v], rows, sem).wait()
            pltpu.sync_copy(rows, outs[t].at[pl.ds(base, BPW)])

    return gather_k


BLK = 512
GRID = B // BLK


def _mlp_body(xr, w0e, w1e, d0, d1, d2, d3, d4, d5,
              w0t, b0, w1t, b1, wltw, wlth, bl, out):
    deep_cat = jnp.concatenate(
        [d0[...], d1[...], d2[...], d3[...], d4[...], d5[...],
         xr[:, WIDE_DIM + N_DEEP:]], axis=1)
    h = jnp.dot(deep_cat, w0t[...], preferred_element_type=jnp.float32) + b0[...]
    h = jnp.maximum(h, 0.0)
    h = jnp.dot(h, w1t[...], preferred_element_type=jnp.float32) + b1[...]
    h = jnp.maximum(h, 0.0)
    wide_cat = jnp.concatenate(
        [w0e[...], w1e[...], xr[:, N_WIDE:WIDE_DIM]], axis=1)
    out[...] = (
        jnp.dot(wide_cat, wltw[...], preferred_element_type=jnp.float32)
        + jnp.dot(h, wlth[...], preferred_element_type=jnp.float32)
        + bl[...]
    )


def _row_spec(d):
    return pl.BlockSpec((BLK, d), lambda i: (i, 0))


def _full_spec(shape):
    return pl.BlockSpec(shape, lambda i: (0,) * len(shape))


def _mlp(xr, w0e, w1e, ds, w0t, b0, w1t, b1, wltw, wlth, bl, interpret=False):
    in_specs = (
        [_row_spec(WIDE_DIM + DEEP_DIM), _row_spec(WD), _row_spec(WD)]
        + [_row_spec(DD)] * N_DEEP
        + [_full_spec(w0t.shape), _full_spec(b0.shape), _full_spec(w1t.shape),
           _full_spec(b1.shape), _full_spec(wltw.shape), _full_spec(wlth.shape),
           _full_spec(bl.shape)]
    )
    return pl.pallas_call(
        _mlp_body,
        grid=(GRID,),
        in_specs=in_specs,
        out_specs=_row_spec(ACTION_DIM),
        out_shape=jax.ShapeDtypeStruct((B, ACTION_DIM), jnp.float32),
        interpret=interpret,
    )(xr, w0e, w1e, *ds, w0t, b0, w1t, b1, wltw, wlth, bl)


def kernel(x, Ew0, Ew1, Ed0, Ed1, Ed2, Ed3, Ed4, Ed5, W0, b0, W1, b1, Wl, bl):
    xi = x.astype(jnp.int32)
    cols = [0, 1] + [WIDE_DIM + i for i in range(N_DEEP)]
    idx_flat = jnp.concatenate([xi[:, c] for c in cols], axis=0)
    gathered = _build_gather()(idx_flat, Ew0, Ew1, Ed0, Ed1, Ed2, Ed3, Ed4, Ed5)
    w0e, w1e = gathered[0], gathered[1]
    ds = gathered[2:]
    wlt = Wl.T
    return _mlp(
        x, w0e, w1e, ds,
        W0.T, b0[None, :], W1.T, b1[None, :],
        wlt[:WIDE_OUT], wlt[WIDE_OUT:], bl[None, :],
    )


# speedup vs baseline: 4.8925x; 4.6909x over previous
import functools

import jax
import jax.numpy as jnp
from jax import lax
from jax.experimental import pallas as pl
from jax.experimental.pallas import tpu as pltpu
from jax.experimental.pallas import tpu_sc as plsc

B = 4096
VOCAB = 100000
WIDE_DIM = 8
DEEP_DIM = 26
N_WIDE = 2
WD = 16
N_DEEP = 6
DD = 32
DEEP_RAW = DEEP_DIM - N_DEEP
WIDE_RAW = WIDE_DIM - N_WIDE
H0, H1 = 256, 128
WIDE_OUT = N_WIDE * WD + WIDE_RAW
Z_DIM = WIDE_OUT + H1
ACTION_DIM = 64

NC, NS = 2, 16
NW = NC * NS
LANES = 16

DEEP_WPT = 4
DEEP_RPW = DD // DEEP_WPT
WIDE_WPT = 4
WIDE_RPW = WD // WIDE_WPT
WIDE_W0 = N_DEEP * DEEP_WPT


@functools.cache
def _build_gather():
    mesh = plsc.VectorSubcoreMesh(
        core_axis_name="c", subcore_axis_name="s", num_cores=NC, num_subcores=NS
    )
    out_type = (
        [jax.ShapeDtypeStruct((WD, B), jnp.float32)] * N_WIDE
        + [jax.ShapeDtypeStruct((DD, B), jnp.float32)] * N_DEEP
    )
    scratch = [
        pltpu.VMEM((VOCAB,), jnp.float32),
        pltpu.VMEM((B,), jnp.int32),
        pltpu.VMEM((B,), jnp.float32),
    ]

    @functools.partial(
        pl.kernel, mesh=mesh, out_type=out_type, scratch_types=scratch,
        compiler_params=pltpu.CompilerParams(
            use_tc_tiling_on_sc=True, needs_layout_passes=False))
    def gather_k(idx8, ewt0, ewt1, edt0, edt1, edt2, edt3, edt4, edt5,
                 gw0, gw1, gd0, gd1, gd2, gd3, gd4, gd5,
                 row_v, idx_v, out_v):
        wid = lax.axis_index("s") * NC + lax.axis_index("c")

        def do_rows(tab, out, idx_row, w_lo, rpw):
            pltpu.sync_copy(idx8.at[idx_row], idx_v)
            lw = wid - w_lo

            def row_body(k, _):
                d = lw * rpw + k
                pltpu.sync_copy(tab.at[d], row_v)

                def gath(i, _):
                    ids = idx_v[pl.ds(i * LANES, LANES)]
                    out_v[pl.ds(i * LANES, LANES)] = plsc.load_gather(
                        row_v, [ids])
                    return 0

                lax.fori_loop(0, B // LANES, gath, 0)
                pltpu.sync_copy(out_v, out.at[d])
                return 0

            lax.fori_loop(0, rpw, row_body, 0)

        deep_tabs = [edt0, edt1, edt2, edt3, edt4, edt5]
        deep_outs = [gd0, gd1, gd2, gd3, gd4, gd5]
        for t in range(N_DEEP):
            w_lo = t * DEEP_WPT

            @pl.when((wid >= w_lo) & (wid < w_lo + DEEP_WPT))
            def _(t=t, w_lo=w_lo):
                do_rows(deep_tabs[t], deep_outs[t], N_WIDE + t, w_lo, DEEP_RPW)

        wide_tabs = [ewt0, ewt1]
        wide_outs = [gw0, gw1]
        for t in range(N_WIDE):
            w_lo = WIDE_W0 + t * WIDE_WPT

            @pl.when((wid >= w_lo) & (wid < w_lo + WIDE_WPT))
            def _(t=t, w_lo=w_lo):
                do_rows(wide_tabs[t], wide_outs[t], t, w_lo, WIDE_RPW)

    return gather_k


BLK = 512
GRID = B // BLK


def _mlp_body(xt, gw0, gw1, gd0, gd1, gd2, gd3, gd4, gd5,
              w0, b0, w1, b1, wl, bl, out):
    dt = jnp.concatenate(
        [gd0[...], gd1[...], gd2[...], gd3[...], gd4[...], gd5[...],
         xt[WIDE_DIM + N_DEEP:, :]], axis=0)
    h = jnp.dot(w0[...], dt, preferred_element_type=jnp.float32) + b0[...]
    h = jnp.maximum(h, 0.0)
    h = jnp.dot(w1[...], h, preferred_element_type=jnp.float32) + b1[...]
    h = jnp.maximum(h, 0.0)
    wt = jnp.concatenate(
        [gw0[...], gw1[...], xt[N_WIDE:WIDE_DIM, :]], axis=0)
    zt = jnp.concatenate([wt, h], axis=0)
    out[...] = jnp.dot(wl[...], zt, preferred_element_type=jnp.float32) + bl[...]


def _col_spec(d):
    return pl.BlockSpec((d, BLK), lambda i: (0, i))


def _full_spec(shape):
    return pl.BlockSpec(shape, lambda i: (0,) * len(shape))


def _mlp(xt, gws, gds, w0, b0, w1, b1, wl, bl, interpret=False):
    in_specs = (
        [_col_spec(WIDE_DIM + DEEP_DIM)]
        + [_col_spec(WD)] * N_WIDE
        + [_col_spec(DD)] * N_DEEP
        + [_full_spec(w0.shape), _full_spec(b0.shape), _full_spec(w1.shape),
           _full_spec(b1.shape), _full_spec(wl.shape), _full_spec(bl.shape)]
    )
    return pl.pallas_call(
        _mlp_body,
        grid=(GRID,),
        in_specs=in_specs,
        out_specs=_col_spec(ACTION_DIM),
        out_shape=jax.ShapeDtypeStruct((ACTION_DIM, B), jnp.float32),
        interpret=interpret,
    )(xt, *gws, *gds, w0, b0, w1, b1, wl, bl)


def kernel(x, Ew0, Ew1, Ed0, Ed1, Ed2, Ed3, Ed4, Ed5, W0, b0, W1, b1, Wl, bl):
    xt = x.T
    idx8 = jnp.concatenate(
        [xt[0:N_WIDE, :], xt[WIDE_DIM:WIDE_DIM + N_DEEP, :]], axis=0
    ).astype(jnp.int32)
    gathered = _build_gather()(
        idx8, Ew0.T, Ew1.T, Ed0.T, Ed1.T, Ed2.T, Ed3.T, Ed4.T, Ed5.T)
    gws = gathered[:N_WIDE]
    gds = gathered[N_WIDE:]
    out_t = _mlp(xt, gws, gds,
                 W0, b0[:, None], W1, b1[:, None], Wl, bl[:, None])
    return out_t.T


# speedup vs baseline: 4.9693x; 1.0157x over previous
import functools

import jax
import jax.numpy as jnp
from jax import lax
from jax.experimental import pallas as pl
from jax.experimental.pallas import tpu as pltpu
from jax.experimental.pallas import tpu_sc as plsc

B = 4096
VOCAB = 100000
WIDE_DIM = 8
DEEP_DIM = 26
N_WIDE = 2
WD = 16
N_DEEP = 6
DD = 32
DEEP_RAW = DEEP_DIM - N_DEEP
WIDE_RAW = WIDE_DIM - N_WIDE
H0, H1 = 256, 128
WIDE_OUT = N_WIDE * WD + WIDE_RAW
Z_DIM = WIDE_OUT + H1
ACTION_DIM = 64

NC, NS = 2, 16
NW = NC * NS
LANES = 16

DEEP_WPT = 4
DEEP_RPW = DD // DEEP_WPT
WIDE_WPT = 4
WIDE_RPW = WD // WIDE_WPT
WIDE_W0 = N_DEEP * DEEP_WPT


@functools.cache
def _build_gather():
    mesh = plsc.VectorSubcoreMesh(
        core_axis_name="c", subcore_axis_name="s", num_cores=NC, num_subcores=NS
    )
    out_type = (
        [jax.ShapeDtypeStruct((WD, B), jnp.float32)] * N_WIDE
        + [jax.ShapeDtypeStruct((DD, B), jnp.float32)] * N_DEEP
    )
    scratch = [
        pltpu.VMEM((VOCAB,), jnp.float32),
        pltpu.VMEM((B,), jnp.float32),
        pltpu.VMEM((B,), jnp.int32),
        pltpu.VMEM((B,), jnp.float32),
    ]

    @functools.partial(
        pl.kernel, mesh=mesh, out_type=out_type, scratch_types=scratch,
        compiler_params=pltpu.CompilerParams(
            use_tc_tiling_on_sc=True, needs_layout_passes=False))
    def gather_k(xt, ewt0, ewt1, edt0, edt1, edt2, edt3, edt4, edt5,
                 gw0, gw1, gd0, gd1, gd2, gd3, gd4, gd5,
                 row_v, xf_v, idx_v, out_v):
        wid = lax.axis_index("s") * NC + lax.axis_index("c")

        def do_rows(tab, out, idx_row, w_lo, rpw):
            pltpu.sync_copy(xt.at[idx_row], xf_v)

            def conv(i, _):
                sl = pl.ds(i * LANES, LANES)
                idx_v[sl] = xf_v[sl].astype(jnp.int32)
                return 0

            lax.fori_loop(0, B // LANES, conv, 0)
            lw = wid - w_lo

            def row_body(k, _):
                d = lw * rpw + k
                pltpu.sync_copy(tab.at[d], row_v)

                def gath(i, _):
                    ids = idx_v[pl.ds(i * LANES, LANES)]
                    out_v[pl.ds(i * LANES, LANES)] = plsc.load_gather(
                        row_v, [ids])
                    return 0

                lax.fori_loop(0, B // LANES, gath, 0)
                pltpu.sync_copy(out_v, out.at[d])
                return 0

            lax.fori_loop(0, rpw, row_body, 0)

        deep_tabs = [edt0, edt1, edt2, edt3, edt4, edt5]
        deep_outs = [gd0, gd1, gd2, gd3, gd4, gd5]
        for t in range(N_DEEP):
            w_lo = t * DEEP_WPT

            @pl.when((wid >= w_lo) & (wid < w_lo + DEEP_WPT))
            def _(t=t, w_lo=w_lo):
                do_rows(deep_tabs[t], deep_outs[t], WIDE_DIM + t, w_lo, DEEP_RPW)

        wide_tabs = [ewt0, ewt1]
        wide_outs = [gw0, gw1]
        for t in range(N_WIDE):
            w_lo = WIDE_W0 + t * WIDE_WPT

            @pl.when((wid >= w_lo) & (wid < w_lo + WIDE_WPT))
            def _(t=t, w_lo=w_lo):
                do_rows(wide_tabs[t], wide_outs[t], t, w_lo, WIDE_RPW)


    return gather_k


BLK = 1024
GRID = B // BLK


def _mlp_body(xt, gw0, gw1, gd0, gd1, gd2, gd3, gd4, gd5,
              w0, b0, w1, b1, wl, bl, out):
    dt = jnp.concatenate(
        [gd0[...], gd1[...], gd2[...], gd3[...], gd4[...], gd5[...],
         xt[WIDE_DIM + N_DEEP:, :]], axis=0)
    h = jnp.dot(w0[...], dt, preferred_element_type=jnp.float32) + b0[...]
    h = jnp.maximum(h, 0.0)
    h = jnp.dot(w1[...], h, preferred_element_type=jnp.float32) + b1[...]
    h = jnp.maximum(h, 0.0)
    wt = jnp.concatenate(
        [gw0[...], gw1[...], xt[N_WIDE:WIDE_DIM, :]], axis=0)
    zt = jnp.concatenate([wt, h], axis=0)
    out[...] = jnp.dot(wl[...], zt, preferred_element_type=jnp.float32) + bl[...]


def _col_spec(d):
    return pl.BlockSpec((d, BLK), lambda i: (0, i))


def _full_spec(shape):
    return pl.BlockSpec(shape, lambda i: (0,) * len(shape))


def _mlp(xt, gws, gds, w0, b0, w1, b1, wl, bl, interpret=False):
    in_specs = (
        [_col_spec(WIDE_DIM + DEEP_DIM)]
        + [_col_spec(WD)] * N_WIDE
        + [_col_spec(DD)] * N_DEEP
        + [_full_spec(w0.shape), _full_spec(b0.shape), _full_spec(w1.shape),
           _full_spec(b1.shape), _full_spec(wl.shape), _full_spec(bl.shape)]
    )
    return pl.pallas_call(
        _mlp_body,
        grid=(GRID,),
        in_specs=in_specs,
        out_specs=_col_spec(ACTION_DIM),
        out_shape=jax.ShapeDtypeStruct((ACTION_DIM, B), jnp.float32),
        interpret=interpret,
    )(xt, *gws, *gds, w0, b0, w1, b1, wl, bl)


def kernel(x, Ew0, Ew1, Ed0, Ed1, Ed2, Ed3, Ed4, Ed5, W0, b0, W1, b1, Wl, bl):
    xt = x.T
    gathered = _build_gather()(
        xt, Ew0.T, Ew1.T, Ed0.T, Ed1.T, Ed2.T, Ed3.T, Ed4.T, Ed5.T)
    gws = gathered[:N_WIDE]
    gds = gathered[N_WIDE:]
    out_t = _mlp(xt, gws, gds,
                 W0, b0[:, None], W1, b1[:, None], Wl, bl[:, None])
    return out_t.T
